# baseline (speedup 1.0000x reference)
import jax
import jax.numpy as jnp
from jax.experimental import pallas as pl

_F32 = jnp.float32
_BF16 = jnp.bfloat16


def _dot(a, b):
    return jnp.dot(a.astype(_BF16), b.astype(_BF16),
                   preferred_element_type=_F32)


def _dot_t(a, b):
    return jax.lax.dot_general(a.astype(_BF16), b.astype(_BF16),
                               (((0,), (0,)), ((), ())),
                               preferred_element_type=_F32)


def _k1(h_ref, x_ref, hb_ref, e0_ref, de_ref):
    @pl.when(pl.program_id(0) == 0)
    def _():
        e0_ref[...] = jnp.zeros_like(e0_ref)
        de_ref[...] = jnp.zeros_like(de_ref)

    h = h_ref[...]
    hb_ref[...] = h.astype(_BF16)
    e0_ref[...] += _dot_t(h, x_ref[...])
    de_ref[...] += jnp.sum(h, axis=0, keepdims=True)


def _k2(h_ref, g0_ref, b0_ref, x_ref, ep_ref):
    @pl.when(pl.program_id(0) == 0)
    def _():
        ep_ref[...] = jnp.zeros_like(ep_ref)

    h = h_ref[...]
    dv = jnp.clip(jnp.sum(h, axis=1, keepdims=True, dtype=_F32), 1e-6, None)
    xb = jnp.maximum(_dot(h, g0_ref[...]) / dv + b0_ref[...], 0.0)
    x_ref[...] = xb.astype(_BF16)
    ep_ref[...] += _dot_t(h, xb)


def _k3(h_ref, ep_ref, wpt_ref, bp_ref, x_ref, s_ref, x2_ref, dv2_ref):
    @pl.when(pl.program_id(0) == 0)
    def _():
        x2_ref[...] = jnp.zeros_like(x2_ref)
        dv2_ref[...] = jnp.zeros_like(dv2_ref)

    h = h_ref[...]
    dv_raw = jnp.sum(h, axis=1, keepdims=True, dtype=_F32)
    dv = jnp.clip(dv_raw, 1e-6, None)
    t = _dot(h, ep_ref[...]) / dv
    logits = _dot(t, wpt_ref[...]) + bp_ref[...]
    mx = jnp.max(logits, axis=1, keepdims=True)
    ex = jnp.exp(logits - mx)
    s = ex / jnp.sum(ex, axis=1, keepdims=True)
    s_ref[...] = s.astype(_BF16)
    x2_ref[...] += _dot_t(s, x_ref[...])
    dv2_ref[...] += _dot_t(s, dv_raw)


def _k4(h_ref, s_ref, x2_ref, e2_ref):
    @pl.when(pl.program_id(0) == 0)
    def _():
        e2_ref[...] = jnp.zeros_like(e2_ref)

    b = _dot(s_ref[...], x2_ref[...])
    e2_ref[...] += _dot_t(h_ref[...], b)


def _k5(h_ref, s_ref, a_ref, xn2_ref):
    @pl.when(pl.program_id(0) == 0)
    def _():
        xn2_ref[...] = jnp.zeros_like(xn2_ref)

    c = _dot(h_ref[...], a_ref[...])
    xn2_ref[...] += _dot_t(s_ref[...], c)


def _row_block(n):
    for bn in (400, 200, 80, 40, 16, 8):
        if n % bn == 0:
            return bn
    return n


def kernel(node_features, incidence, W0, b0, Wp, bp, W1, b1, Wr, br):
    n, m = incidence.shape
    d = node_features.shape[1]
    k = Wp.shape[0]
    bn = _row_block(n)
    nb = n // bn
    grid = (nb,)

    def row_spec(cols):
        return pl.BlockSpec((bn, cols), lambda i: (i, 0))

    def acc_spec(rows, cols):
        return pl.BlockSpec((rows, cols), lambda i: (0, 0))

    f32 = jnp.float32
    bf16 = jnp.bfloat16

    hb, e0_raw, de_row = pl.pallas_call(
        _k1,
        grid=grid,
        in_specs=[row_spec(m), row_spec(d)],
        out_specs=[row_spec(m), acc_spec(m, d), acc_spec(1, m)],
        out_shape=[jax.ShapeDtypeStruct((n, m), bf16),
                   jax.ShapeDtypeStruct((m, d), f32),
                   jax.ShapeDtypeStruct((1, m), f32)],
    )(incidence, node_features)

    de = jnp.clip(de_row[0], 1e-6, None)
    g0 = _dot(e0_raw / de[:, None], W0.T)

    x, ep_raw = pl.pallas_call(
        _k2,
        grid=grid,
        in_specs=[row_spec(m), acc_spec(m, d), acc_spec(1, d)],
        out_specs=[row_spec(d), acc_spec(m, d)],
        out_shape=[jax.ShapeDtypeStruct((n, d), bf16),
                   jax.ShapeDtypeStruct((m, d), f32)],
    )(hb, g0, b0[None, :])

    ep = (ep_raw / de[:, None]).astype(bf16)

    s, x2, dv2_col = pl.pallas_call(
        _k3,
        grid=grid,
        in_specs=[row_spec(m), acc_spec(m, d), acc_spec(d, k),
                  acc_spec(1, k), row_spec(d)],
        out_specs=[row_spec(k), acc_spec(k, d), acc_spec(k, 1)],
        out_shape=[jax.ShapeDtypeStruct((n, k), bf16),
                   jax.ShapeDtypeStruct((k, d), f32),
                   jax.ShapeDtypeStruct((k, 1), f32)],
    )(hb, ep, Wp.T, bp[None, :], x)

    e2_raw = pl.pallas_call(
        _k4,
        grid=grid,
        in_specs=[row_spec(m), row_spec(k), acc_spec(k, d)],
        out_specs=acc_spec(m, d),
        out_shape=jax.ShapeDtypeStruct((m, d), f32),
    )(hb, s, x2)

    a = (_dot(e2_raw / de[:, None], W1.T)).astype(bf16)

    xn2 = pl.pallas_call(
        _k5,
        grid=grid,
        in_specs=[row_spec(m), row_spec(k), acc_spec(m, d)],
        out_specs=acc_spec(k, d),
        out_shape=jax.ShapeDtypeStruct((k, d), f32),
    )(hb, s, a)

    dv2 = jnp.clip(dv2_col[:, 0], 1e-6, None)
    x3 = jnp.maximum(xn2 / dv2[:, None] + b1[None, :], 0.0)
    return _dot(Wr, jnp.mean(x3, axis=0)) + br


# speedup vs baseline: 1.1497x; 1.1497x over previous
import jax
import jax.numpy as jnp
from jax.experimental import pallas as pl

_F32 = jnp.float32
_BF16 = jnp.bfloat16


def _dot(a, b):
    return jnp.dot(a.astype(_BF16), b.astype(_BF16),
                   preferred_element_type=_F32)


def _dot_t(a, b):
    return jax.lax.dot_general(a.astype(_BF16), b.astype(_BF16),
                               (((0,), (0,)), ((), ())),
                               preferred_element_type=_F32)


def _k1(h_ref, x_ref, hb_ref, e0_ref, de_ref):
    @pl.when(pl.program_id(0) == 0)
    def _():
        e0_ref[...] = jnp.zeros_like(e0_ref)
        de_ref[...] = jnp.zeros_like(de_ref)

    h = h_ref[...]
    hb_ref[...] = h.astype(_BF16)
    e0_ref[...] += _dot_t(h, x_ref[...])
    de_ref[...] += jnp.sum(h, axis=0, keepdims=True)


def _k2(h_ref, g0_ref, b0_ref, x_ref, ep_ref):
    @pl.when(pl.program_id(0) == 0)
    def _():
        ep_ref[...] = jnp.zeros_like(ep_ref)

    h = h_ref[...]
    dv = jnp.clip(jnp.sum(h, axis=1, keepdims=True, dtype=_F32), 1e-6, None)
    xb = jnp.maximum(_dot(h, g0_ref[...]) / dv + b0_ref[...], 0.0)
    x_ref[...] = xb.astype(_BF16)
    ep_ref[...] += _dot_t(h, xb)


def _k3(h_ref, ep_ref, wpt_ref, bp_ref, x_ref, s_ref, x2_ref, dv2_ref):
    @pl.when(pl.program_id(0) == 0)
    def _():
        x2_ref[...] = jnp.zeros_like(x2_ref)
        dv2_ref[...] = jnp.zeros_like(dv2_ref)

    h = h_ref[...]
    dv_raw = jnp.sum(h, axis=1, keepdims=True, dtype=_F32)
    dv = jnp.clip(dv_raw, 1e-6, None)
    t = _dot(h, ep_ref[...]) / dv
    logits = _dot(t, wpt_ref[...]) + bp_ref[...]
    mx = jnp.max(logits, axis=1, keepdims=True)
    ex = jnp.exp(logits - mx)
    s = ex / jnp.sum(ex, axis=1, keepdims=True)
    s_ref[...] = s.astype(_BF16)
    x2_ref[...] += _dot_t(s, x_ref[...])
    dv2_ref[...] += _dot_t(s, dv_raw)


def _k4(h_ref, s_ref, x2_ref, e2_ref):
    @pl.when(pl.program_id(0) == 0)
    def _():
        e2_ref[...] = jnp.zeros_like(e2_ref)

    b = _dot(s_ref[...], x2_ref[...])
    e2_ref[...] += _dot_t(h_ref[...], b)


def _k5(h_ref, s_ref, a_ref, xn2_ref):
    @pl.when(pl.program_id(0) == 0)
    def _():
        xn2_ref[...] = jnp.zeros_like(xn2_ref)

    c = _dot(h_ref[...], a_ref[...])
    xn2_ref[...] += _dot_t(s_ref[...], c)


def _row_block(n):
    for bn in (2000, 1000, 400, 200, 80, 40, 16, 8):
        if n % bn == 0:
            return bn
    return n


def kernel(node_features, incidence, W0, b0, Wp, bp, W1, b1, Wr, br):
    n, m = incidence.shape
    d = node_features.shape[1]
    k = Wp.shape[0]
    bn = _row_block(n)
    nb = n // bn
    grid = (nb,)

    def row_spec(cols):
        return pl.BlockSpec((bn, cols), lambda i: (i, 0))

    def acc_spec(rows, cols):
        return pl.BlockSpec((rows, cols), lambda i: (0, 0))

    f32 = jnp.float32
    bf16 = jnp.bfloat16

    hb, e0_raw, de_row = pl.pallas_call(
        _k1,
        grid=grid,
        in_specs=[row_spec(m), row_spec(d)],
        out_specs=[row_spec(m), acc_spec(m, d), acc_spec(1, m)],
        out_shape=[jax.ShapeDtypeStruct((n, m), bf16),
                   jax.ShapeDtypeStruct((m, d), f32),
                   jax.ShapeDtypeStruct((1, m), f32)],
    )(incidence, node_features)

    de = jnp.clip(de_row[0], 1e-6, None)
    g0 = _dot(e0_raw / de[:, None], W0.T)

    x, ep_raw = pl.pallas_call(
        _k2,
        grid=grid,
        in_specs=[row_spec(m), acc_spec(m, d), acc_spec(1, d)],
        out_specs=[row_spec(d), acc_spec(m, d)],
        out_shape=[jax.ShapeDtypeStruct((n, d), bf16),
                   jax.ShapeDtypeStruct((m, d), f32)],
    )(hb, g0, b0[None, :])

    ep = (ep_raw / de[:, None]).astype(bf16)

    s, x2, dv2_col = pl.pallas_call(
        _k3,
        grid=grid,
        in_specs=[row_spec(m), acc_spec(m, d), acc_spec(d, k),
                  acc_spec(1, k), row_spec(d)],
        out_specs=[row_spec(k), acc_spec(k, d), acc_spec(k, 1)],
        out_shape=[jax.ShapeDtypeStruct((n, k), bf16),
                   jax.ShapeDtypeStruct((k, d), f32),
                   jax.ShapeDtypeStruct((k, 1), f32)],
    )(hb, ep, Wp.T, bp[None, :], x)

    e2_raw = pl.pallas_call(
        _k4,
        grid=grid,
        in_specs=[row_spec(m), row_spec(k), acc_spec(k, d)],
        out_specs=acc_spec(m, d),
        out_shape=jax.ShapeDtypeStruct((m, d), f32),
    )(hb, s, x2)

    a = (_dot(e2_raw / de[:, None], W1.T)).astype(bf16)

    xn2 = pl.pallas_call(
        _k5,
        grid=grid,
        in_specs=[row_spec(m), row_spec(k), acc_spec(m, d)],
        out_specs=acc_spec(k, d),
        out_shape=jax.ShapeDtypeStruct((k, d), f32),
    )(hb, s, a)

    dv2 = jnp.clip(dv2_col[:, 0], 1e-6, None)
    x3 = jnp.maximum(xn2 / dv2[:, None] + b1[None, :], 0.0)
    return _dot(Wr, jnp.mean(x3, axis=0)) + br


# speedup vs baseline: 1.3422x; 1.1674x over previous
import jax
import jax.numpy as jnp
from jax.experimental import pallas as pl
from jax.experimental.pallas import tpu as pltpu

_F32 = jnp.float32
_BF16 = jnp.bfloat16


def _dot(a, b):
    return jnp.dot(a.astype(_BF16), b.astype(_BF16),
                   preferred_element_type=_F32)


def _dot_t(a, b):
    return jax.lax.dot_general(a.astype(_BF16), b.astype(_BF16),
                               (((0,), (0,)), ((), ())),
                               preferred_element_type=_F32)


def _ka(ht_ref, x_ref, hbt_ref, e0_ref, de_ref):
    ht = ht_ref[...]
    hbt_ref[...] = ht.astype(_BF16)
    e0_ref[...] = _dot(ht, x_ref[...])
    de_ref[...] = jnp.sum(ht, axis=1, keepdims=True)


def _kb(ht_ref, g0_ref, b0_ref, x_ref, dv_ref, xacc, dvacc):
    i = pl.program_id(0)

    @pl.when(i == 0)
    def _():
        xacc[...] = jnp.zeros_like(xacc)
        dvacc[...] = jnp.zeros_like(dvacc)

    ht = ht_ref[...]
    xacc[...] += _dot_t(ht, g0_ref[...])
    dvacc[...] += _dot_t(ht, jnp.ones_like(ht[:, :1], _F32))

    @pl.when(i == pl.num_programs(0) - 1)
    def _():
        dv = jnp.clip(dvacc[...], 1e-6, None)
        dv_ref[...] = dv
        x = jnp.maximum(xacc[...] / dv + b0_ref[...], 0.0)
        x_ref[...] = x.astype(_BF16)


def _kc(ht_ref, x_ref, de_ref, t_ref, tacc):
    i = pl.program_id(0)

    @pl.when(i == 0)
    def _():
        tacc[...] = jnp.zeros_like(tacc)

    ht = ht_ref[...]
    ep = _dot(ht, x_ref[...]) / de_ref[...]
    tacc[...] += _dot_t(ht, ep)

    @pl.when(i == pl.num_programs(0) - 1)
    def _():
        t_ref[...] = tacc[...].astype(_BF16)


def _kd(t_ref, dv_ref, wpt_ref, bp_ref, x_ref, s_ref, x2_ref, dv2_ref):
    @pl.when(pl.program_id(0) == 0)
    def _():
        x2_ref[...] = jnp.zeros_like(x2_ref)
        dv2_ref[...] = jnp.zeros_like(dv2_ref)

    dv = jnp.clip(dv_ref[...], 1e-6, None)
    t = t_ref[...].astype(_F32) / dv
    logits = _dot(t, wpt_ref[...]) + bp_ref[...]
    mx = jnp.max(logits, axis=1, keepdims=True)
    ex = jnp.exp(logits - mx)
    s = ex / jnp.sum(ex, axis=1, keepdims=True)
    s_ref[...] = s.astype(_BF16)
    x2_ref[...] += _dot_t(s, x_ref[...])
    dv2_ref[...] += _dot_t(s, dv_ref[...])


def _ke(s_ref, x2_ref, b_ref):
    b_ref[...] = _dot(s_ref[...], x2_ref[...]).astype(_BF16)


def _kf(ht_ref, b_ref, de_ref, w1t_ref, c_ref, cacc):
    i = pl.program_id(0)

    @pl.when(i == 0)
    def _():
        cacc[...] = jnp.zeros_like(cacc)

    ht = ht_ref[...]
    e2 = _dot(ht, b_ref[...]) / de_ref[...]
    a = _dot(e2, w1t_ref[...])
    cacc[...] += _dot_t(ht, a)

    @pl.when(i == pl.num_programs(0) - 1)
    def _():
        c_ref[...] = cacc[...].astype(_BF16)


def _kg(s_ref, c_ref, xn2_ref):
    @pl.when(pl.program_id(0) == 0)
    def _():
        xn2_ref[...] = jnp.zeros_like(xn2_ref)

    xn2_ref[...] += _dot_t(s_ref[...], c_ref[...])


def _pick_block(dim, cands):
    for b in cands:
        if dim % b == 0:
            return b
    return dim


def kernel(node_features, incidence, W0, b0, Wp, bp, W1, b1, Wr, br):
    n, m = incidence.shape
    d = node_features.shape[1]
    k = Wp.shape[0]
    bm = _pick_block(m, (200, 100, 40, 8))
    bm2 = _pick_block(m, (400, 200, 100, 40, 8))
    bn = _pick_block(n, (1000, 400, 200, 80, 8))
    f32, bf16 = jnp.float32, jnp.bfloat16

    ht = incidence.T

    def edge_spec(block):
        return pl.BlockSpec((block, n), lambda i: (i, 0))

    def edge_small(block, cols):
        return pl.BlockSpec((block, cols), lambda i: (i, 0))

    def node_spec(cols, block=None):
        return pl.BlockSpec((block or bn, cols), lambda i: (i, 0))

    def full_spec(rows, cols):
        return pl.BlockSpec((rows, cols), lambda i: (0, 0))

    hbt, e0, de_raw = pl.pallas_call(
        _ka,
        grid=(m // bm,),
        in_specs=[edge_spec(bm), full_spec(n, d)],
        out_specs=[edge_spec(bm), edge_small(bm, d), edge_small(bm, 1)],
        out_shape=[jax.ShapeDtypeStruct((m, n), bf16),
                   jax.ShapeDtypeStruct((m, d), f32),
                   jax.ShapeDtypeStruct((m, 1), f32)],
    )(ht, node_features)

    de = jnp.clip(de_raw, 1e-6, None)
    g0 = _dot(e0 / de, W0.T)

    x, dv = pl.pallas_call(
        _kb,
        grid=(m // bm2,),
        in_specs=[edge_spec(bm2), edge_small(bm2, d), full_spec(1, d)],
        out_specs=[full_spec(n, d), full_spec(n, 1)],
        out_shape=[jax.ShapeDtypeStruct((n, d), bf16),
                   jax.ShapeDtypeStruct((n, 1), f32)],
        scratch_shapes=[pltpu.VMEM((n, d), f32), pltpu.VMEM((n, 1), f32)],
    )(hbt, g0, b0[None, :])

    t = pl.pallas_call(
        _kc,
        grid=(m // bm2,),
        in_specs=[edge_spec(bm2), full_spec(n, d), edge_small(bm2, 1)],
        out_specs=full_spec(n, d),
        out_shape=jax.ShapeDtypeStruct((n, d), bf16),
        scratch_shapes=[pltpu.VMEM((n, d), f32)],
    )(hbt, x, de)

    s, x2, dv2_col = pl.pallas_call(
        _kd,
        grid=(n // bn,),
        in_specs=[node_spec(d), node_spec(1), full_spec(d, k),
                  full_spec(1, k), node_spec(d)],
        out_specs=[node_spec(k), full_spec(k, d), full_spec(k, 1)],
        out_shape=[jax.ShapeDtypeStruct((n, k), bf16),
                   jax.ShapeDtypeStruct((k, d), f32),
                   jax.ShapeDtypeStruct((k, 1), f32)],
    )(t, dv, Wp.T, bp[None, :], x)

    b_mat = pl.pallas_call(
        _ke,
        grid=(n // bn,),
        in_specs=[node_spec(k), full_spec(k, d)],
        out_specs=node_spec(d),
        out_shape=jax.ShapeDtypeStruct((n, d), bf16),
    )(s, x2)

    c_mat = pl.pallas_call(
        _kf,
        grid=(m // bm2,),
        in_specs=[edge_spec(bm2), full_spec(n, d), edge_small(bm2, 1),
                  full_spec(d, d)],
        out_specs=full_spec(n, d),
        out_shape=jax.ShapeDtypeStruct((n, d), bf16),
        scratch_shapes=[pltpu.VMEM((n, d), f32)],
    )(hbt, b_mat, de, W1.T)

    xn2 = pl.pallas_call(
        _kg,
        grid=(n // bn,),
        in_specs=[node_spec(k), node_spec(d)],
        out_specs=full_spec(k, d),
        out_shape=jax.ShapeDtypeStruct((k, d), f32),
    )(s, c_mat)

    dv2 = jnp.clip(dv2_col[:, 0], 1e-6, None)
    x3 = jnp.maximum(xn2 / dv2[:, None] + b1[None, :], 0.0)
    return _dot(Wr, jnp.mean(x3, axis=0)) + br


# speedup vs baseline: 1.3479x; 1.0042x over previous
import jax
import jax.numpy as jnp
from jax.experimental import pallas as pl
from jax.experimental.pallas import tpu as pltpu

_F32 = jnp.float32
_BF16 = jnp.bfloat16


def _dot(a, b):
    return jnp.dot(a.astype(_BF16), b.astype(_BF16),
                   preferred_element_type=_F32)


def _dot_t(a, b):
    return jax.lax.dot_general(a.astype(_BF16), b.astype(_BF16),
                               (((0,), (0,)), ((), ())),
                               preferred_element_type=_F32)


def _ka(ht_ref, x_ref, hbt_ref, e0_ref, de_ref):
    ht = ht_ref[...]
    hbt_ref[...] = ht.astype(_BF16)
    e0_ref[...] = _dot(ht, x_ref[...])
    de_ref[...] = jnp.sum(ht, axis=1, keepdims=True)


def _kb(ht_ref, g0_ref, b0_ref, x_ref, dv_ref, xacc, dvacc):
    i = pl.program_id(0)

    @pl.when(i == 0)
    def _():
        xacc[...] = jnp.zeros_like(xacc)
        dvacc[...] = jnp.zeros_like(dvacc)

    ht = ht_ref[...]
    xacc[...] += _dot_t(ht, g0_ref[...])
    dvacc[...] += _dot_t(ht, jnp.ones_like(ht[:, :1], _F32))

    @pl.when(i == pl.num_programs(0) - 1)
    def _():
        dv = jnp.clip(dvacc[...], 1e-6, None)
        dv_ref[...] = dv
        x = jnp.maximum(xacc[...] / dv + b0_ref[...], 0.0)
        x_ref[...] = x.astype(_BF16)


def _kc(ht_ref, x_ref, de_ref, t_ref, tacc):
    i = pl.program_id(0)

    @pl.when(i == 0)
    def _():
        tacc[...] = jnp.zeros_like(tacc)

    ht = ht_ref[...]
    ep = _dot(ht, x_ref[...]) / de_ref[...]
    tacc[...] += _dot_t(ht, ep)

    @pl.when(i == pl.num_programs(0) - 1)
    def _():
        t_ref[...] = tacc[...].astype(_BF16)


def _kd(t_ref, dv_ref, wpt_ref, bp_ref, x_ref, s_ref, x2_ref, dv2_ref):
    @pl.when(pl.program_id(0) == 0)
    def _():
        x2_ref[...] = jnp.zeros_like(x2_ref)
        dv2_ref[...] = jnp.zeros_like(dv2_ref)

    dv = jnp.clip(dv_ref[...], 1e-6, None)
    t = t_ref[...].astype(_F32) / dv
    logits = _dot(t, wpt_ref[...]) + bp_ref[...]
    mx = jnp.max(logits, axis=1, keepdims=True)
    ex = jnp.exp(logits - mx)
    s = ex / jnp.sum(ex, axis=1, keepdims=True)
    s_ref[...] = s.astype(_BF16)
    x2_ref[...] += _dot_t(s, x_ref[...])
    dv2_ref[...] += _dot_t(s, dv_ref[...])


def _ke(s_ref, x2_ref, b_ref):
    b_ref[...] = _dot(s_ref[...], x2_ref[...]).astype(_BF16)


def _kf(ht_ref, b_ref, de_ref, c_ref, cacc):
    i = pl.program_id(0)

    @pl.when(i == 0)
    def _():
        cacc[...] = jnp.zeros_like(cacc)

    ht = ht_ref[...]
    a = _dot(ht, b_ref[...]) / de_ref[...]
    cacc[...] += _dot_t(ht, a)

    @pl.when(i == pl.num_programs(0) - 1)
    def _():
        c_ref[...] = cacc[...].astype(_BF16)


def _kg(s_ref, c_ref, xn2_ref):
    @pl.when(pl.program_id(0) == 0)
    def _():
        xn2_ref[...] = jnp.zeros_like(xn2_ref)

    xn2_ref[...] += _dot_t(s_ref[...], c_ref[...])


def _pick_block(dim, cands):
    for b in cands:
        if dim % b == 0:
            return b
    return dim


def kernel(node_features, incidence, W0, b0, Wp, bp, W1, b1, Wr, br):
    n, m = incidence.shape
    d = node_features.shape[1]
    k = Wp.shape[0]
    bm = _pick_block(m, (200, 100, 40, 8))
    bm2 = _pick_block(m, (400, 200, 100, 40, 8))
    bn = _pick_block(n, (2000, 1000, 400, 200, 80, 8))
    f32, bf16 = jnp.float32, jnp.bfloat16

    ht = incidence.T

    def edge_spec(block):
        return pl.BlockSpec((block, n), lambda i: (i, 0))

    def edge_small(block, cols):
        return pl.BlockSpec((block, cols), lambda i: (i, 0))

    def node_spec(cols, block=None):
        return pl.BlockSpec((block or bn, cols), lambda i: (i, 0))

    def full_spec(rows, cols):
        return pl.BlockSpec((rows, cols), lambda i: (0, 0))

    hbt, e0, de_raw = pl.pallas_call(
        _ka,
        grid=(m // bm,),
        in_specs=[edge_spec(bm), full_spec(n, d)],
        out_specs=[edge_spec(bm), edge_small(bm, d), edge_small(bm, 1)],
        out_shape=[jax.ShapeDtypeStruct((m, n), bf16),
                   jax.ShapeDtypeStruct((m, d), f32),
                   jax.ShapeDtypeStruct((m, 1), f32)],
    )(ht, node_features)

    de = jnp.clip(de_raw, 1e-6, None)
    g0 = _dot(e0 / de, W0.T)

    x, dv = pl.pallas_call(
        _kb,
        grid=(m // bm2,),
        in_specs=[edge_spec(bm2), edge_small(bm2, d), full_spec(1, d)],
        out_specs=[full_spec(n, d), full_spec(n, 1)],
        out_shape=[jax.ShapeDtypeStruct((n, d), bf16),
                   jax.ShapeDtypeStruct((n, 1), f32)],
        scratch_shapes=[pltpu.VMEM((n, d), f32), pltpu.VMEM((n, 1), f32)],
    )(hbt, g0, b0[None, :])

    t = pl.pallas_call(
        _kc,
        grid=(m // bm2,),
        in_specs=[edge_spec(bm2), full_spec(n, d), edge_small(bm2, 1)],
        out_specs=full_spec(n, d),
        out_shape=jax.ShapeDtypeStruct((n, d), bf16),
        scratch_shapes=[pltpu.VMEM((n, d), f32)],
    )(hbt, x, de)

    s, x2, dv2_col = pl.pallas_call(
        _kd,
        grid=(n // bn,),
        in_specs=[node_spec(d), node_spec(1), full_spec(d, k),
                  full_spec(1, k), node_spec(d)],
        out_specs=[node_spec(k), full_spec(k, d), full_spec(k, 1)],
        out_shape=[jax.ShapeDtypeStruct((n, k), bf16),
                   jax.ShapeDtypeStruct((k, d), f32),
                   jax.ShapeDtypeStruct((k, 1), f32)],
    )(t, dv, Wp.T, bp[None, :], x)

    x2w = _dot(x2, W1.T)

    b_mat = pl.pallas_call(
        _ke,
        grid=(n // bn,),
        in_specs=[node_spec(k), full_spec(k, d)],
        out_specs=node_spec(d),
        out_shape=jax.ShapeDtypeStruct((n, d), bf16),
    )(s, x2w)

    c_mat = pl.pallas_call(
        _kf,
        grid=(m // bm2,),
        in_specs=[edge_spec(bm2), full_spec(n, d), edge_small(bm2, 1)],
        out_specs=full_spec(n, d),
        out_shape=jax.ShapeDtypeStruct((n, d), bf16),
        scratch_shapes=[pltpu.VMEM((n, d), f32)],
    )(hbt, b_mat, de)

    xn2 = pl.pallas_call(
        _kg,
        grid=(n // bn,),
        in_specs=[node_spec(k), node_spec(d)],
        out_specs=full_spec(k, d),
        out_shape=jax.ShapeDtypeStruct((k, d), f32),
    )(s, c_mat)

    dv2 = jnp.clip(dv2_col[:, 0], 1e-6, None)
    x3 = jnp.maximum(xn2 / dv2[:, None] + b1[None, :], 0.0)
    return _dot(Wr, jnp.mean(x3, axis=0)) + br


# speedup vs baseline: 1.7773x; 1.3186x over previous
import jax
import jax.numpy as jnp
from jax.experimental import pallas as pl
from jax.experimental.pallas import tpu as pltpu

_F32 = jnp.float32
_BF16 = jnp.bfloat16


def _dot(a, b):
    return jnp.dot(a.astype(_BF16), b.astype(_BF16),
                   preferred_element_type=_F32)


def _dot_t(a, b):
    return jax.lax.dot_general(a.astype(_BF16), b.astype(_BF16),
                               (((0,), (0,)), ((), ())),
                               preferred_element_type=_F32)


def _ka(ht_ref, x_ref, hbt_ref, e0_ref, de_ref):
    ht = ht_ref[...]
    hbt_ref[...] = ht.astype(_BF16)
    e0_ref[...] = _dot(ht, x_ref[...])
    de_ref[...] = jnp.sum(ht, axis=1, keepdims=True)


def _make_mega(bm, bn, m, n, d, k):
    nsteps = m // bm
    assert nsteps == n // bn

    def mega(ht_ref, g0_ref, de_ref, b0_ref, wpt_ref, bp_ref, w1t_ref,
             xn2_ref, dv2_ref,
             s_scr, x_scr, acc_scr, dv_scr, x2_scr):
        bp_scr = x_scr
        p = pl.program_id(0)
        s = pl.program_id(1)
        last = nsteps - 1
        er = pl.ds(s * bm, bm)
        nr = pl.ds(s * bn, bn)

        @pl.when(p == 0)
        def _phase_b():
            @pl.when(s == 0)
            def _():
                acc_scr[...] = jnp.zeros_like(acc_scr)
                dv_scr[...] = jnp.zeros_like(dv_scr)

            ht = ht_ref[...]
            acc_scr[...] += _dot_t(ht, g0_ref[er, :])
            dv_scr[...] += _dot_t(ht, jnp.ones_like(ht[:, :1], _F32))

            @pl.when(s == last)
            def _():
                dv = jnp.clip(dv_scr[...], 1e-6, None)
                dv_scr[...] = dv
                x_scr[...] = jnp.maximum(
                    acc_scr[...] / dv + b0_ref[...], 0.0).astype(_BF16)

        @pl.when(p == 1)
        def _phase_c():
            @pl.when(s == 0)
            def _():
                acc_scr[...] = jnp.zeros_like(acc_scr)

            ht = ht_ref[...]
            ep = _dot(ht, x_scr[...]) / de_ref[er, :]
            acc_scr[...] += _dot_t(ht, ep)

        @pl.when(p == 2)
        def _phase_d():
            @pl.when(s == 0)
            def _():
                x2_scr[...] = jnp.zeros_like(x2_scr)
                dv2_ref[...] = jnp.zeros_like(dv2_ref)

            dvb = dv_scr[nr, :]
            t = acc_scr[nr, :] / dvb
            logits = _dot(t, wpt_ref[...]) + bp_ref[...]
            mx = jnp.max(logits, axis=1, keepdims=True)
            ex = jnp.exp(logits - mx)
            sm = ex / jnp.sum(ex, axis=1, keepdims=True)
            s_scr[nr, :] = sm.astype(_BF16)
            x2_scr[...] += _dot_t(sm, x_scr[nr, :])
            dv2_ref[...] += _dot_t(sm, dvb)

        @pl.when(p == 3)
        def _phase_e():
            @pl.when(s == 0)
            def _():
                x2_scr[...] = _dot(x2_scr[...], w1t_ref[...])

            bp_scr[nr, :] = _dot(s_scr[nr, :], x2_scr[...]).astype(_BF16)

        @pl.when(p == 4)
        def _phase_f():
            @pl.when(s == 0)
            def _():
                acc_scr[...] = jnp.zeros_like(acc_scr)

            ht = ht_ref[...]
            a = _dot(ht, bp_scr[...]) / de_ref[er, :]
            acc_scr[...] += _dot_t(ht, a)

        @pl.when(p == 5)
        def _phase_g():
            @pl.when(s == 0)
            def _():
                xn2_ref[...] = jnp.zeros_like(xn2_ref)

            xn2_ref[...] += _dot_t(s_scr[nr, :], acc_scr[nr, :])

    return mega


def _pick_block(dim, cands):
    for b in cands:
        if dim % b == 0:
            return b
    return dim


def kernel(node_features, incidence, W0, b0, Wp, bp, W1, b1, Wr, br):
    n, m = incidence.shape
    d = node_features.shape[1]
    k = Wp.shape[0]
    bm = _pick_block(m, (200, 100, 40, 8))
    bm2 = _pick_block(m, (200, 100, 40, 8))
    bn = (n * bm2) // m
    f32, bf16 = jnp.float32, jnp.bfloat16

    ht = incidence.T
    nsteps = m // bm2

    hbt, e0, de_raw = pl.pallas_call(
        _ka,
        grid=(m // bm,),
        in_specs=[pl.BlockSpec((bm, n), lambda i: (i, 0)),
                  pl.BlockSpec((n, d), lambda i: (0, 0))],
        out_specs=[pl.BlockSpec((bm, n), lambda i: (i, 0)),
                   pl.BlockSpec((bm, d), lambda i: (i, 0)),
                   pl.BlockSpec((bm, 1), lambda i: (i, 0))],
        out_shape=[jax.ShapeDtypeStruct((m, n), bf16),
                   jax.ShapeDtypeStruct((m, d), f32),
                   jax.ShapeDtypeStruct((m, 1), f32)],
    )(ht, node_features)

    de = jnp.clip(de_raw, 1e-6, None)
    g0 = _dot(e0 / de, W0.T)

    def ht_index(p, s):
        use = (p == 0) | (p == 1) | (p == 4)
        return (jnp.where(use, s, nsteps - 1), 0)

    def const_index(p, s):
        return (0, 0)

    mega = _make_mega(bm2, bn, m, n, d, k)
    xn2, dv2_col = pl.pallas_call(
        mega,
        grid=(6, nsteps),
        in_specs=[pl.BlockSpec((bm2, n), ht_index),
                  pl.BlockSpec((m, d), const_index),
                  pl.BlockSpec((m, 1), const_index),
                  pl.BlockSpec((1, d), const_index),
                  pl.BlockSpec((d, k), const_index),
                  pl.BlockSpec((1, k), const_index),
                  pl.BlockSpec((d, d), const_index)],
        out_specs=[pl.BlockSpec((k, d), const_index),
                   pl.BlockSpec((k, 1), const_index)],
        out_shape=[jax.ShapeDtypeStruct((k, d), f32),
                   jax.ShapeDtypeStruct((k, 1), f32)],
        scratch_shapes=[pltpu.VMEM((n, k), bf16),
                        pltpu.VMEM((n, d), bf16),
                        pltpu.VMEM((n, d), f32),
                        pltpu.VMEM((n, 1), f32),
                        pltpu.VMEM((k, d), f32)],
    )(hbt, g0, de, b0[None, :], Wp.T, bp[None, :], W1.T)

    dv2 = jnp.clip(dv2_col[:, 0], 1e-6, None)
    x3 = jnp.maximum(xn2 / dv2[:, None] + b1[None, :], 0.0)
    return _dot(Wr, jnp.mean(x3, axis=0)) + br


# speedup vs baseline: 1.8307x; 1.0300x over previous
import jax
import jax.numpy as jnp
from jax.experimental import pallas as pl
from jax.experimental.pallas import tpu as pltpu

_F32 = jnp.float32
_BF16 = jnp.bfloat16


def _dot(a, b):
    return jnp.dot(a.astype(_BF16), b.astype(_BF16),
                   preferred_element_type=_F32)


def _dot_t(a, b):
    return jax.lax.dot_general(a.astype(_BF16), b.astype(_BF16),
                               (((0,), (0,)), ((), ())),
                               preferred_element_type=_F32)


def _ka(ht_ref, x_ref, w0t_ref, hbt_ref, g0_ref, de_ref):
    ht = ht_ref[...]
    hbt_ref[...] = ht.astype(_BF16)
    e0 = _dot(ht, x_ref[...])
    de = jnp.clip(jnp.sum(ht, axis=1, keepdims=True), 1e-6, None)
    de_ref[...] = de
    g0_ref[...] = _dot(e0 / de, w0t_ref[...])


def _make_mega(bm, bn, m, n, d, k):
    nsteps = m // bm
    assert nsteps == n // bn

    def mega(ht_ref, g0_ref, de_ref, b0_ref, wpt_ref, bp_ref, w1t_ref,
             b1_ref, wrt_ref, br_ref, out_ref,
             s_scr, x_scr, acc_scr, dv_scr, x2_scr, dv2_scr):
        bp_scr = x_scr
        xn2_scr = x2_scr
        p = pl.program_id(0)
        s = pl.program_id(1)
        last = nsteps - 1
        er = pl.ds(s * bm, bm)
        nr = pl.ds(s * bn, bn)

        @pl.when(p == 0)
        def _phase_b():
            @pl.when(s == 0)
            def _():
                acc_scr[...] = jnp.zeros_like(acc_scr)
                dv_scr[...] = jnp.zeros_like(dv_scr)

            ht = ht_ref[...]
            acc_scr[...] += _dot_t(ht, g0_ref[er, :])
            dv_scr[...] += _dot_t(ht, jnp.ones_like(ht[:, :1], _F32))

            @pl.when(s == last)
            def _():
                dv = jnp.clip(dv_scr[...], 1e-6, None)
                dv_scr[...] = dv
                x_scr[...] = jnp.maximum(
                    acc_scr[...] / dv + b0_ref[...], 0.0).astype(_BF16)

        @pl.when(p == 1)
        def _phase_c():
            @pl.when(s == 0)
            def _():
                acc_scr[...] = jnp.zeros_like(acc_scr)

            ht = ht_ref[...]
            ep = _dot(ht, x_scr[...]) / de_ref[er, :]
            acc_scr[...] += _dot_t(ht, ep)

        @pl.when(p == 2)
        def _phase_d():
            @pl.when(s == 0)
            def _():
                x2_scr[...] = jnp.zeros_like(x2_scr)
                dv2_scr[...] = jnp.zeros_like(dv2_scr)

            dvb = dv_scr[nr, :]
            t = acc_scr[nr, :] / dvb
            logits = _dot(t, wpt_ref[...]) + bp_ref[...]
            mx = jnp.max(logits, axis=1, keepdims=True)
            ex = jnp.exp(logits - mx)
            sm = ex / jnp.sum(ex, axis=1, keepdims=True)
            s_scr[nr, :] = sm.astype(_BF16)
            x2_scr[...] += _dot_t(sm, x_scr[nr, :])
            dv2_scr[...] += _dot_t(sm, dvb)

        @pl.when(p == 3)
        def _phase_e():
            @pl.when(s == 0)
            def _():
                x2_scr[...] = _dot(x2_scr[...], w1t_ref[...])

            bp_scr[nr, :] = _dot(s_scr[nr, :], x2_scr[...]).astype(_BF16)

        @pl.when(p == 4)
        def _phase_f():
            @pl.when(s == 0)
            def _():
                acc_scr[...] = jnp.zeros_like(acc_scr)

            ht = ht_ref[...]
            a = _dot(ht, bp_scr[...]) / de_ref[er, :]
            acc_scr[...] += _dot_t(ht, a)

        @pl.when(p == 5)
        def _phase_g():
            @pl.when(s == 0)
            def _():
                xn2_scr[...] = jnp.zeros_like(xn2_scr)

            xn2_scr[...] += _dot_t(s_scr[nr, :], acc_scr[nr, :])

            @pl.when(s == last)
            def _():
                dv2 = jnp.clip(dv2_scr[...], 1e-6, None)
                x3 = jnp.maximum(xn2_scr[...] / dv2 + b1_ref[...], 0.0)
                emb = jnp.sum(x3, axis=0, keepdims=True) * (1.0 / k)
                out_ref[...] = _dot(emb, wrt_ref[...]) + br_ref[...]

    return mega


def _pick_block(dim, cands):
    for b in cands:
        if dim % b == 0:
            return b
    return dim


def kernel(node_features, incidence, W0, b0, Wp, bp, W1, b1, Wr, br):
    n, m = incidence.shape
    d = node_features.shape[1]
    k = Wp.shape[0]
    bm = _pick_block(m, (200, 100, 40, 8))
    bm2 = _pick_block(m, (200, 100, 40, 8))
    bn = (n * bm2) // m
    f32, bf16 = jnp.float32, jnp.bfloat16

    ht = incidence.T
    nsteps = m // bm2

    hbt, g0, de = pl.pallas_call(
        _ka,
        grid=(m // bm,),
        in_specs=[pl.BlockSpec((bm, n), lambda i: (i, 0)),
                  pl.BlockSpec((n, d), lambda i: (0, 0)),
                  pl.BlockSpec((d, d), lambda i: (0, 0))],
        out_specs=[pl.BlockSpec((bm, n), lambda i: (i, 0)),
                   pl.BlockSpec((bm, d), lambda i: (i, 0)),
                   pl.BlockSpec((bm, 1), lambda i: (i, 0))],
        out_shape=[jax.ShapeDtypeStruct((m, n), bf16),
                   jax.ShapeDtypeStruct((m, d), f32),
                   jax.ShapeDtypeStruct((m, 1), f32)],
    )(ht, node_features, W0.T)

    def ht_index(p, s):
        use = (p == 0) | (p == 1) | (p == 4)
        return (jnp.where(use, s, nsteps - 1), 0)

    def const_index(p, s):
        return (0, 0)

    mega = _make_mega(bm2, bn, m, n, d, k)
    out = pl.pallas_call(
        mega,
        grid=(6, nsteps),
        in_specs=[pl.BlockSpec((bm2, n), ht_index),
                  pl.BlockSpec((m, d), const_index),
                  pl.BlockSpec((m, 1), const_index),
                  pl.BlockSpec((1, d), const_index),
                  pl.BlockSpec((d, k), const_index),
                  pl.BlockSpec((1, k), const_index),
                  pl.BlockSpec((d, d), const_index),
                  pl.BlockSpec((1, d), const_index),
                  pl.BlockSpec((d, Wr.shape[0]), const_index),
                  pl.BlockSpec((1, Wr.shape[0]), const_index)],
        out_specs=pl.BlockSpec((1, Wr.shape[0]), const_index),
        out_shape=jax.ShapeDtypeStruct((1, Wr.shape[0]), f32),
        scratch_shapes=[pltpu.VMEM((n, k), bf16),
                        pltpu.VMEM((n, d), bf16),
                        pltpu.VMEM((n, d), f32),
                        pltpu.VMEM((n, 1), f32),
                        pltpu.VMEM((k, d), f32),
                        pltpu.VMEM((k, 1), f32)],
    )(hbt, g0, de, b0[None, :], Wp.T, bp[None, :], W1.T,
      b1[None, :], Wr.T, br[None, :])

    return out[0]


# speedup vs baseline: 1.9836x; 1.0835x over previous
import jax
import jax.numpy as jnp
from jax.experimental import pallas as pl
from jax.experimental.pallas import tpu as pltpu

_F32 = jnp.float32
_BF16 = jnp.bfloat16


def _dot(a, b):
    return jnp.dot(a.astype(_BF16), b.astype(_BF16),
                   preferred_element_type=_F32)


def _dot_t(a, b):
    return jax.lax.dot_general(a.astype(_BF16), b.astype(_BF16),
                               (((0,), (0,)), ((), ())),
                               preferred_element_type=_F32)


def _conv0(ht_ref, xin_ref, w0t_ref, b0_ref,
           hbt_ref, de_ref, x_ref, dv_ref, xacc, dvacc):
    i = pl.program_id(0)

    @pl.when(i == 0)
    def _():
        xacc[...] = jnp.zeros_like(xacc)
        dvacc[...] = jnp.zeros_like(dvacc)

    ht = ht_ref[...]
    hbt_ref[...] = ht.astype(_BF16)
    e0 = _dot(ht, xin_ref[...])
    de = jnp.clip(jnp.sum(ht, axis=1, keepdims=True), 1e-6, None)
    de_ref[...] = de
    g0 = _dot(e0 / de, w0t_ref[...])
    xacc[...] += _dot_t(ht, g0)
    dvacc[...] += _dot_t(ht, jnp.ones_like(ht[:, :1], _F32))

    @pl.when(i == pl.num_programs(0) - 1)
    def _():
        dv = jnp.clip(dvacc[...], 1e-6, None)
        dv_ref[...] = dv
        x_ref[...] = jnp.maximum(
            xacc[...] / dv + b0_ref[...], 0.0).astype(_BF16)


def _make_mega(bm, bn, m, n, d, k):
    nsteps = m // bm
    assert nsteps == n // bn

    def mega(ht_ref, x_ref, dv_ref, de_ref, wpt_ref, bp_ref, w1t_ref,
             b1_ref, wrt_ref, br_ref, out_ref,
             s_scr, bp_scr, acc_scr, x2_scr, dv2_scr):
        xn2_scr = x2_scr
        p = pl.program_id(0)
        s = pl.program_id(1)
        last = nsteps - 1
        er = pl.ds(s * bm, bm)
        nr = pl.ds(s * bn, bn)

        @pl.when(p == 0)
        def _phase_c():
            @pl.when(s == 0)
            def _():
                acc_scr[...] = jnp.zeros_like(acc_scr)

            ht = ht_ref[...]
            ep = _dot(ht, x_ref[...]) / de_ref[er, :]
            acc_scr[...] += _dot_t(ht, ep)

        @pl.when(p == 1)
        def _phase_d():
            @pl.when(s == 0)
            def _():
                x2_scr[...] = jnp.zeros_like(x2_scr)
                dv2_scr[...] = jnp.zeros_like(dv2_scr)

            dvb = dv_ref[nr, :]
            t = acc_scr[nr, :] / dvb
            logits = _dot(t, wpt_ref[...]) + bp_ref[...]
            mx = jnp.max(logits, axis=1, keepdims=True)
            ex = jnp.exp(logits - mx)
            sm = ex / jnp.sum(ex, axis=1, keepdims=True)
            s_scr[nr, :] = sm.astype(_BF16)
            x2_scr[...] += _dot_t(sm, x_ref[nr, :])
            dv2_scr[...] += _dot_t(sm, dvb)

        @pl.when(p == 2)
        def _phase_e():
            @pl.when(s == 0)
            def _():
                x2_scr[...] = _dot(x2_scr[...], w1t_ref[...])

            bp_scr[nr, :] = _dot(s_scr[nr, :], x2_scr[...]).astype(_BF16)

        @pl.when(p == 3)
        def _phase_f():
            @pl.when(s == 0)
            def _():
                acc_scr[...] = jnp.zeros_like(acc_scr)

            ht = ht_ref[...]
            a = _dot(ht, bp_scr[...]) / de_ref[er, :]
            acc_scr[...] += _dot_t(ht, a)

        @pl.when(p == 4)
        def _phase_g():
            @pl.when(s == 0)
            def _():
                xn2_scr[...] = jnp.zeros_like(xn2_scr)

            xn2_scr[...] += _dot_t(s_scr[nr, :], acc_scr[nr, :])

            @pl.when(s == last)
            def _():
                dv2 = jnp.clip(dv2_scr[...], 1e-6, None)
                x3 = jnp.maximum(xn2_scr[...] / dv2 + b1_ref[...], 0.0)
                emb = jnp.sum(x3, axis=0, keepdims=True) * (1.0 / k)
                out_ref[...] = _dot(emb, wrt_ref[...]) + br_ref[...]

    return mega


def _pick_block(dim, cands):
    for b in cands:
        if dim % b == 0:
            return b
    return dim


def kernel(node_features, incidence, W0, b0, Wp, bp, W1, b1, Wr, br):
    n, m = incidence.shape
    d = node_features.shape[1]
    k = Wp.shape[0]
    o = Wr.shape[0]
    bm = _pick_block(m, (200, 100, 40, 8))
    bm2 = _pick_block(m, (200, 100, 40, 8))
    bn = (n * bm2) // m
    f32, bf16 = jnp.float32, jnp.bfloat16

    ht = incidence.T
    nsteps = m // bm2

    hbt, de, x, dv = pl.pallas_call(
        _conv0,
        grid=(m // bm,),
        in_specs=[pl.BlockSpec((bm, n), lambda i: (i, 0)),
                  pl.BlockSpec((n, d), lambda i: (0, 0)),
                  pl.BlockSpec((d, d), lambda i: (0, 0)),
                  pl.BlockSpec((1, d), lambda i: (0, 0))],
        out_specs=[pl.BlockSpec((bm, n), lambda i: (i, 0)),
                   pl.BlockSpec((bm, 1), lambda i: (i, 0)),
                   pl.BlockSpec((n, d), lambda i: (0, 0)),
                   pl.BlockSpec((n, 1), lambda i: (0, 0))],
        out_shape=[jax.ShapeDtypeStruct((m, n), bf16),
                   jax.ShapeDtypeStruct((m, 1), f32),
                   jax.ShapeDtypeStruct((n, d), bf16),
                   jax.ShapeDtypeStruct((n, 1), f32)],
        scratch_shapes=[pltpu.VMEM((n, d), f32), pltpu.VMEM((n, 1), f32)],
    )(ht, node_features, W0.T, b0[None, :])

    def ht_index(p, s):
        use = (p == 0) | (p == 3)
        return (jnp.where(use, s, nsteps - 1), 0)

    def const_index(p, s):
        return (0, 0)

    mega = _make_mega(bm2, bn, m, n, d, k)
    out = pl.pallas_call(
        mega,
        grid=(5, nsteps),
        in_specs=[pl.BlockSpec((bm2, n), ht_index),
                  pl.BlockSpec((n, d), const_index),
                  pl.BlockSpec((n, 1), const_index),
                  pl.BlockSpec((m, 1), const_index),
                  pl.BlockSpec((d, k), const_index),
                  pl.BlockSpec((1, k), const_index),
                  pl.BlockSpec((d, d), const_index),
                  pl.BlockSpec((1, d), const_index),
                  pl.BlockSpec((d, o), const_index),
                  pl.BlockSpec((1, o), const_index)],
        out_specs=pl.BlockSpec((1, o), const_index),
        out_shape=jax.ShapeDtypeStruct((1, o), f32),
        scratch_shapes=[pltpu.VMEM((n, k), bf16),
                        pltpu.VMEM((n, d), bf16),
                        pltpu.VMEM((n, d), f32),
                        pltpu.VMEM((k, d), f32),
                        pltpu.VMEM((k, 1), f32)],
    )(hbt, x, dv, de, Wp.T, bp[None, :], W1.T,
      b1[None, :], Wr.T, br[None, :])

    return out[0]
